# Initial kernel scaffold; baseline (speedup 1.0000x reference)
import jax
import jax.numpy as jnp
from jax.experimental import pallas as pl


def kernel(x_p, x_d, edge_attr_p, edge_attr_d, edge_index_p, edge_index_d, batch_p, batch_d, params):
    raise NotImplementedError("write your pallas kernel here")


# speedup vs baseline: 18.7136x; 18.7136x over previous
import functools

import jax
import jax.numpy as jnp
from jax import lax
from jax.experimental import pallas as pl
from jax.experimental.pallas import tpu as pltpu
from jax.experimental.pallas import tpu_sc as plsc

N = 10000
E = 320000
D = 128
B = 64
NP_PAD = 10240
M = 2 * NP_PAD
NC, NS, LANES = 2, 16, 16
NW = NC * NS
K = 128
EE = 2 * E
CH = -(-EE // (NW * K))
EEP = CH * NW * K
PAD_ROW = NP_PAD - 1
RB = 256
NBLK = M // RB
NBP = NP_PAD // RB
ROWS_PER_TILE = M // NS
F32 = jnp.float32

_MESH = dict(core_axis_name="c", subcore_axis_name="s",
             num_cores=NC, num_subcores=NS)


def _dot(a, b):
    return lax.dot(a, b, precision=lax.Precision.HIGHEST,
                   preferred_element_type=F32)


def _dot_t(a, b):
    return lax.dot_general(a, b, (((0,), (0,)), ((), ())),
                           precision=lax.Precision.HIGHEST,
                           preferred_element_type=F32)


def _fill_rows(ref, nrows, ncols, value):
    vec = jnp.full((LANES,), value, F32)
    for r in range(nrows):
        for c in range(ncols // LANES):
            ref[r, pl.ds(c * LANES, LANES)] = vec


def _zero_my_acc_slice(zsrc, acc, si, rows_per_copy):
    ncopy = ROWS_PER_TILE // rows_per_copy
    base = si * ROWS_PER_TILE
    for t in range(ncopy):
        pltpu.sync_copy(zsrc, acc.at[pl.ds(base + t * rows_per_copy,
                                           rows_per_copy)])


@functools.lru_cache(maxsize=None)
def _get_sc_degree():
    @functools.partial(
        pl.kernel,
        out_type=jax.ShapeDtypeStruct((NC, M, LANES), F32),
        mesh=plsc.VectorSubcoreMesh(**_MESH),
        compiler_params=pltpu.CompilerParams(use_tc_tiling_on_sc=False),
        scratch_types=[
            pltpu.VMEM((CH, K), jnp.int32),
            pltpu.VMEM((K, LANES), F32),
            pltpu.VMEM((K, LANES), F32),
            pltpu.VMEM_SHARED((M, LANES), F32),
        ],
    )
    def deg(dst_hbm, out_hbm, dst_v, ones_v, zeros_v, acc):
        ci = lax.axis_index("c")
        si = lax.axis_index("s")
        wid = si * NC + ci
        _fill_rows(ones_v, K, LANES, 1.0)
        _fill_rows(zeros_v, K, LANES, 0.0)
        _zero_my_acc_slice(zeros_v, acc, si, K)
        pltpu.sync_copy(dst_hbm.at[wid], dst_v)
        plsc.subcore_barrier()

        def body(j, carry):
            pltpu.sync_copy(ones_v, acc.at[dst_v.at[j]], add=True)
            return carry

        lax.fori_loop(0, CH, body, 0)
        plsc.subcore_barrier()
        base = si * ROWS_PER_TILE
        pltpu.sync_copy(acc.at[pl.ds(base, ROWS_PER_TILE)],
                        out_hbm.at[ci, pl.ds(base, ROWS_PER_TILE)])

    return deg


@functools.lru_cache(maxsize=None)
def _get_sc_aggregate(feat):
    @functools.partial(
        pl.kernel,
        out_type=jax.ShapeDtypeStruct((NC, M, feat), F32),
        mesh=plsc.VectorSubcoreMesh(**_MESH),
        compiler_params=pltpu.CompilerParams(use_tc_tiling_on_sc=False),
        scratch_types=[
            pltpu.VMEM((CH, K), jnp.int32),
            pltpu.VMEM((CH, K), jnp.int32),
            pltpu.VMEM((K, feat), F32),
            pltpu.VMEM_SHARED((M, feat), F32),
            pltpu.SemaphoreType.DMA,
        ],
    )
    def agg(hw_hbm, src_hbm, dst_hbm, out_hbm, src_v, dst_v, gbuf, acc, sem):
        ci = lax.axis_index("c")
        si = lax.axis_index("s")
        wid = si * NC + ci
        _fill_rows(gbuf, K, feat, 0.0)
        _zero_my_acc_slice(gbuf, acc, si, K)
        pltpu.sync_copy(src_hbm.at[wid], src_v)
        pltpu.sync_copy(dst_hbm.at[wid], dst_v)
        plsc.subcore_barrier()

        def body(j, carry):
            pltpu.async_copy(hw_hbm.at[src_v.at[j]], gbuf, sem).wait()
            pltpu.sync_copy(gbuf, acc.at[dst_v.at[j]], add=True)
            return carry

        lax.fori_loop(0, CH, body, 0)
        plsc.subcore_barrier()
        base = si * ROWS_PER_TILE
        pltpu.sync_copy(acc.at[pl.ds(base, ROWS_PER_TILE)],
                        out_hbm.at[ci, pl.ds(base, ROWS_PER_TILE)])

    return agg


def _k1_body(x_ref, dacc_ref, wp1, wd1, ga1w, gb1w, ga1b, gb1b,
             ga2w, gb2w, ga2b, gb2b,
             hw_ref, dinv_ref, g_ref, gmp_ref, gmd_ref):
    i = pl.program_id(0)
    is_p = i < NBP
    deg = dacc_ref[0, :, 0:1] + dacc_ref[1, :, 0:1]
    dinv = lax.rsqrt(deg + 1.0)
    xb = x_ref[...]
    w1 = jnp.where(is_p, wp1[...], wd1[...])
    hw_ref[...] = _dot(xb, w1) * dinv
    dinv_ref[...] = dinv
    g1w = jnp.where(is_p, ga1w[...], gb1w[...])
    g1b = jnp.where(is_p, ga1b[...], gb1b[...])
    g2w = jnp.where(is_p, ga2w[...], gb2w[...])
    g2b = jnp.where(is_p, ga2b[...], gb2b[...])
    t = jnp.maximum(_dot(xb, g1w) + g1b, 0.0)
    g = _dot(t, g2w) + g2b
    g_ref[...] = g
    bm = jnp.max(g, keepdims=True)

    @pl.when(i == 0)
    def _():
        gmp_ref[...] = bm
        gmd_ref[...] = bm - 1.0

    @pl.when((i > 0) & is_p)
    def _():
        gmp_ref[...] = jnp.maximum(gmp_ref[...], bm)

    @pl.when(i == NBP)
    def _():
        gmd_ref[...] = bm

    @pl.when(i > NBP)
    def _():
        gmd_ref[...] = jnp.maximum(gmd_ref[...], bm)


def _tc_k1(x, degacc, p):
    return pl.pallas_call(
        _k1_body,
        grid=(NBLK,),
        in_specs=[
            pl.BlockSpec((RB, D), lambda i: (i, 0)),
            pl.BlockSpec((NC, RB, LANES), lambda i: (0, i, 0)),
        ] + [pl.BlockSpec(w.shape, lambda i: (0, 0))
             for w in (p["Wp1"], p["Wd1"], p["Ga1"], p["Gb1"],
                       p["ga1r"], p["gb1r"], p["Ga2"], p["Gb2"],
                       p["ga2r"], p["gb2r"])],
        out_specs=[
            pl.BlockSpec((RB, 32), lambda i: (i, 0)),
            pl.BlockSpec((RB, 1), lambda i: (i, 0)),
            pl.BlockSpec((RB, 1), lambda i: (i, 0)),
            pl.BlockSpec((1, 1), lambda i: (0, 0)),
            pl.BlockSpec((1, 1), lambda i: (0, 0)),
        ],
        out_shape=[
            jax.ShapeDtypeStruct((M, 32), F32),
            jax.ShapeDtypeStruct((M, 1), F32),
            jax.ShapeDtypeStruct((M, 1), F32),
            jax.ShapeDtypeStruct((1, 1), F32),
            jax.ShapeDtypeStruct((1, 1), F32),
        ],
    )(x, degacc, p["Wp1"], p["Wd1"], p["Ga1"], p["Gb1"], p["ga1r"],
      p["gb1r"], p["Ga2"], p["Gb2"], p["ga2r"], p["gb2r"])


def _kmid_body(acc_ref, hwp_ref, dinv_ref, bp_ref, bd_ref, wp_ref, wd_ref,
               out_ref):
    i = pl.program_id(0)
    is_p = i < NBP
    dinv = dinv_ref[...]
    a = acc_ref[0] + acc_ref[1] + hwp_ref[...]
    b = jnp.where(is_p, bp_ref[...], bd_ref[...])
    h = jnp.maximum(dinv * a + b, 0.0)
    w = jnp.where(is_p, wp_ref[...], wd_ref[...])
    out_ref[...] = _dot(h, w) * dinv


def _tc_mid(acc, hwp, dinv, bp, bd, wp, wd):
    fin, fout = wp.shape
    return pl.pallas_call(
        _kmid_body,
        grid=(NBLK,),
        in_specs=[
            pl.BlockSpec((NC, RB, fin), lambda i: (0, i, 0)),
            pl.BlockSpec((RB, fin), lambda i: (i, 0)),
            pl.BlockSpec((RB, 1), lambda i: (i, 0)),
            pl.BlockSpec((1, fin), lambda i: (0, 0)),
            pl.BlockSpec((1, fin), lambda i: (0, 0)),
            pl.BlockSpec((fin, fout), lambda i: (0, 0)),
            pl.BlockSpec((fin, fout), lambda i: (0, 0)),
        ],
        out_specs=pl.BlockSpec((RB, fout), lambda i: (i, 0)),
        out_shape=jax.ShapeDtypeStruct((M, fout), F32),
    )(acc, hwp, dinv, bp, bd, wp, wd)


def _k7_body(acc_ref, hwp_ref, dinv_ref, bp_ref, bd_ref, g_ref,
             gmp_ref, gmd_ref, x_ref, batch_ref,
             sh_ref, sden_ref, scnt_ref, sx_ref):
    i = pl.program_id(0)
    is_p = i < NBP
    a = acc_ref[0] + acc_ref[1] + hwp_ref[...]
    b = jnp.where(is_p, bp_ref[...], bd_ref[...])
    h3 = jnp.maximum(dinv_ref[...] * a + b, 0.0)
    gm = jnp.where(is_p, gmp_ref[...], gmd_ref[...])
    ge = jnp.exp(g_ref[...] - gm)
    iota = lax.broadcasted_iota(jnp.int32, (1, B), 1).astype(F32)
    oh = (batch_ref[...] == iota).astype(F32)
    sh = _dot_t(oh, h3)
    sden = _dot_t(oh, ge)
    scnt = _dot_t(oh, jnp.ones((RB, 1), F32))
    sx = _dot_t(oh, ge * x_ref[...])

    @pl.when(i % NBP == 0)
    def _():
        sh_ref[0] = sh
        sden_ref[0] = sden
        scnt_ref[0] = scnt
        sx_ref[0] = sx

    @pl.when(i % NBP != 0)
    def _():
        sh_ref[0] += sh
        sden_ref[0] += sden
        scnt_ref[0] += scnt
        sx_ref[0] += sx


def _tc_pool(acc3, hw3p, dinv, bp3, bd3, g, gmp, gmd, x, batch):
    return pl.pallas_call(
        _k7_body,
        grid=(NBLK,),
        in_specs=[
            pl.BlockSpec((NC, RB, 16), lambda i: (0, i, 0)),
            pl.BlockSpec((RB, 16), lambda i: (i, 0)),
            pl.BlockSpec((RB, 1), lambda i: (i, 0)),
            pl.BlockSpec((1, 16), lambda i: (0, 0)),
            pl.BlockSpec((1, 16), lambda i: (0, 0)),
            pl.BlockSpec((RB, 1), lambda i: (i, 0)),
            pl.BlockSpec((1, 1), lambda i: (0, 0)),
            pl.BlockSpec((1, 1), lambda i: (0, 0)),
            pl.BlockSpec((RB, D), lambda i: (i, 0)),
            pl.BlockSpec((RB, 1), lambda i: (i, 0)),
        ],
        out_specs=[
            pl.BlockSpec((1, B, 16), lambda i: (i // NBP, 0, 0)),
            pl.BlockSpec((1, B, 1), lambda i: (i // NBP, 0, 0)),
            pl.BlockSpec((1, B, 1), lambda i: (i // NBP, 0, 0)),
            pl.BlockSpec((1, B, D), lambda i: (i // NBP, 0, 0)),
        ],
        out_shape=[
            jax.ShapeDtypeStruct((2, B, 16), F32),
            jax.ShapeDtypeStruct((2, B, 1), F32),
            jax.ShapeDtypeStruct((2, B, 1), F32),
            jax.ShapeDtypeStruct((2, B, D), F32),
        ],
    )(acc3, hw3p, dinv, bp3, bd3, g, gmp, gmd, x, batch)


def _k9_body(sh_ref, sden_ref, scnt_ref, sx_ref, l1w_ref, l1b_ref,
             l2w_ref, l2b_ref, out_ref):
    pp = sh_ref[0] / jnp.maximum(scnt_ref[0], 1.0)
    pd = sh_ref[1] / jnp.maximum(scnt_ref[1], 1.0)
    ap = sx_ref[0] / jnp.maximum(sden_ref[0], 1e-12)
    ad = sx_ref[1] / jnp.maximum(sden_ref[1], 1e-12)
    z = (_dot(pp, l1w_ref[0:16]) + _dot(pd, l1w_ref[16:32])
         + _dot(ap, l1w_ref[32:160]) + _dot(ad, l1w_ref[160:288])
         + l1b_ref[...])
    z = jnp.maximum(z, 0.0)
    out_ref[...] = _dot(z, l2w_ref[...]) + l2b_ref[...]


def _tc_final(sh, sden, scnt, sx, l1w, l1b, l2w, l2b):
    return pl.pallas_call(
        _k9_body,
        out_shape=jax.ShapeDtypeStruct((B, 1), F32),
    )(sh, sden, scnt, sx, l1w, l1b, l2w, l2b)


def kernel(x_p, x_d, edge_attr_p, edge_attr_d, edge_index_p, edge_index_d,
           batch_p, batch_d, params):
    del edge_attr_p, edge_attr_d
    p = dict(params)
    for k in ("ga1", "gb1", "ga2", "gb2", "bp1", "bp2", "bp3",
              "bd1", "bd2", "bd3", "l1", "l2"):
        p[k + "r"] = p[k].reshape(1, -1).astype(F32)

    idx_dtype = edge_index_p.dtype
    pad_n = NP_PAD - N
    zrows = jnp.zeros((pad_n, D), F32)
    x = jnp.concatenate([x_p, zrows, x_d, zrows], axis=0)

    epad = jnp.full((EEP - EE,), PAD_ROW, idx_dtype)
    src = jnp.concatenate([edge_index_p[0], edge_index_d[0] + NP_PAD, epad])
    dst = jnp.concatenate([edge_index_p[1], edge_index_d[1] + NP_PAD, epad])
    src = src.astype(jnp.int32).reshape(NW, CH, K)
    dst = dst.astype(jnp.int32).reshape(NW, CH, K)

    bpad = jnp.full((pad_n,), B, batch_p.dtype)
    batch = jnp.concatenate([batch_p, bpad, batch_d, bpad])
    batch = batch.astype(F32).reshape(M, 1)

    degacc = _get_sc_degree()(dst)
    hw1, dinv, g, gmp, gmd = _tc_k1(x, degacc, p)
    acc1 = _get_sc_aggregate(32)(hw1, src, dst)
    hw2 = _tc_mid(acc1, hw1, dinv, p["bp1r"], p["bd1r"], p["Wp2"], p["Wd2"])
    acc2 = _get_sc_aggregate(16)(hw2, src, dst)
    hw3 = _tc_mid(acc2, hw2, dinv, p["bp2r"], p["bd2r"], p["Wp3"], p["Wd3"])
    acc3 = _get_sc_aggregate(16)(hw3, src, dst)
    sh, sden, scnt, sx = _tc_pool(acc3, hw3, dinv, p["bp3r"], p["bd3r"],
                                  g, gmp, gmd, x, batch)
    return _tc_final(sh, sden, scnt, sx, p["L1"], p["l1r"], p["L2"], p["l2r"])
